# Initial kernel scaffold; baseline (speedup 1.0000x reference)
import jax
import jax.numpy as jnp
from jax.experimental import pallas as pl


def kernel(input_ids, labels, table):
    raise NotImplementedError("write your pallas kernel here")


# speedup vs baseline: 1.5945x; 1.5945x over previous
import math

import jax
import jax.numpy as jnp
from jax import lax
from jax.experimental import pallas as pl
from jax.experimental.pallas import tpu as pltpu
from jax.experimental.pallas import tpu_sc as plsc

VOCAB = 32000
D_MODEL = 2048
HEAD_DIM = 128
MAX_LEN = 4096
THETA = 10000.0
B = 4
S = 4096

NEG_MIN = float(jnp.finfo(jnp.float32).min)


_NC = 2
_NS = 16
_NW = _NC * _NS
_N_TOK = B * S
_PER_W = _N_TOK // _NW
_CH = 16
_NCH = _PER_W // _CH
_NBUF = 3


def _sc_gather_body(table_hbm, ids_hbm, out_hbm, idx_v, rows_v, gsem, osem):
    wid = lax.axis_index("s") * _NC + lax.axis_index("c")
    base = wid * _PER_W

    pltpu.sync_copy(ids_hbm.at[wid], idx_v)

    def start_gather(c, b):
        pltpu.async_copy(table_hbm.at[idx_v.at[c]], rows_v.at[b], gsem)

    for b in range(_NBUF):
        start_gather(b, b)

    def body(c, carry):
        b = lax.rem(c, _NBUF)
        pltpu.make_async_copy(table_hbm.at[idx_v.at[c]], rows_v.at[b], gsem).wait()
        out_copy = pltpu.make_async_copy(
            rows_v.at[b], out_hbm.at[pl.ds(base + c * _CH, _CH)], osem)
        out_copy.start()
        out_copy.wait()
        @pl.when(c + _NBUF < _NCH)
        def _():
            start_gather(c + _NBUF, b)
        return carry

    lax.fori_loop(0, _NCH, body, 0, unroll=False)


def _sc_gather(table, ids3):
    kern = pl.kernel(
        _sc_gather_body,
        out_type=jax.ShapeDtypeStruct((_N_TOK, D_MODEL), jnp.float32),
        mesh=plsc.VectorSubcoreMesh(core_axis_name="c", subcore_axis_name="s"),
        scratch_types=[
            pltpu.VMEM((_NCH, _CH), jnp.int32),
            pltpu.VMEM((_NBUF, _CH, D_MODEL), jnp.float32),
            pltpu.SemaphoreType.DMA,
            pltpu.SemaphoreType.DMA,
        ],
    )
    return kern(table, ids3)


_MBLK = 512


def _mask_body(o_ref):
    i = pl.program_id(0)
    rows = lax.broadcasted_iota(jnp.int32, (_MBLK, S), 0) + i * _MBLK
    cols = lax.broadcasted_iota(jnp.int32, (_MBLK, S), 1)
    o_ref[...] = jnp.where(cols > rows, NEG_MIN, 0.0).astype(jnp.float32)


def _make_mask():
    return pl.pallas_call(
        _mask_body,
        grid=(S // _MBLK,),
        out_specs=pl.BlockSpec((_MBLK, S), lambda i: (i, 0)),
        out_shape=jax.ShapeDtypeStruct((S, S), jnp.float32),
    )()


_HD2 = HEAD_DIM // 2


def _freqs_body(cos_ref, sin_ref):
    t = lax.broadcasted_iota(jnp.int32, (MAX_LEN, _HD2), 0).astype(jnp.float32)
    j = lax.broadcasted_iota(jnp.int32, (MAX_LEN, _HD2), 1).astype(jnp.float32)
    inv = jnp.exp(j * (-2.0 / HEAD_DIM * math.log(THETA)))
    f = t * inv
    cos_ref[...] = jnp.cos(f)
    sin_ref[...] = jnp.sin(f)


def _make_freqs():
    return pl.pallas_call(
        _freqs_body,
        out_shape=(
            jax.ShapeDtypeStruct((MAX_LEN, _HD2), jnp.float32),
            jax.ShapeDtypeStruct((MAX_LEN, _HD2), jnp.float32),
        ),
    )()


def kernel(input_ids, labels, table):
    ids3 = input_ids.reshape(_NW, _NCH, _CH).astype(jnp.int32)
    hidden = _sc_gather(table, ids3).reshape(B, S, D_MODEL)
    mask = _make_mask().reshape(1, 1, S, S)
    cos, sin = _make_freqs()
    freqs_cis = lax.complex(cos, sin)
    return (hidden, freqs_cis, mask, labels)
